# baseline (speedup 1.0000x reference)
import functools

import jax
import jax.numpy as jnp
from jax.experimental import pallas as pl
from jax.experimental.pallas import tpu as pltpu
from jax.experimental.pallas import tpu_sc as plsc

N = 50000
D = 512
H = 256
G = 128

B = 1000
NB = N // B

_LANES = 16
_KB = 13
_CHUNK = _KB * 128
_NPAD = 32 * _CHUNK


def _k1_body(x_ref, b_ref, wcat_ref, bcat_ref, w2_ref,
             scores_ref, m_ref, emb_ref,
             m_sc, s_sc, acc_sc):
    i = pl.program_id(0)

    @pl.when(i == 0)
    def _init():
        m_sc[...] = jnp.full((1, G), -jnp.inf, jnp.float32)
        s_sc[...] = jnp.zeros((1, G), jnp.float32)
        acc_sc[...] = jnp.zeros((G, D), jnp.float32)

    y = jnp.dot(x_ref[...].astype(jnp.bfloat16), wcat_ref[...],
                preferred_element_type=jnp.float32)
    y = jnp.maximum(y + bcat_ref[...], 0.0)
    g1 = y[:, :H]
    f = y[:, H:]
    score_col = jnp.dot(g1.astype(jnp.bfloat16), w2_ref[...],
                        preferred_element_type=jnp.float32)
    scores_ref[0, :, :] = score_col

    seg = b_ref[0, 0, :]
    onehot = seg[:, None] == jax.lax.broadcasted_iota(jnp.int32, (B, G), 1)
    m_old = m_sc[...][0]
    m_blk = jnp.max(jnp.where(onehot, score_col, -jnp.inf), axis=0)
    m_new = jnp.maximum(m_old, m_blk)
    scale = jnp.where(m_old > -jnp.inf, jnp.exp(m_old - m_new), 0.0)
    e = jnp.where(onehot, jnp.exp(score_col - m_new[None, :]), 0.0)
    s_sc[...] = (s_sc[...][0] * scale + jnp.sum(e, axis=0))[None, :]
    acc_sc[...] = acc_sc[...] * scale[:, None] + jax.lax.dot_general(
        e.astype(jnp.bfloat16), f.astype(jnp.bfloat16),
        (((0,), (0,)), ((), ())), preferred_element_type=jnp.float32)
    m_sc[...] = m_new[None, :]

    @pl.when(i == NB - 1)
    def _fin():
        sden = s_sc[...]
        m_ref[...] = m_sc[...] + jnp.log(sden)
        scol = sden[0][:, None]
        emb_ref[...] = jnp.where(scol > 0.0, acc_sc[...] / scol, 0.0)


def _run_k1(x, batch, wcat, bcat, w2):
    return pl.pallas_call(
        _k1_body,
        grid=(NB,),
        in_specs=[
            pl.BlockSpec((B, D), lambda i: (i, 0)),
            pl.BlockSpec((1, 1, B), lambda i: (i, 0, 0)),
            pl.BlockSpec((D, H + D), lambda i: (0, 0)),
            pl.BlockSpec((1, H + D), lambda i: (0, 0)),
            pl.BlockSpec((H, 1), lambda i: (0, 0)),
        ],
        out_specs=[
            pl.BlockSpec((1, B, 1), lambda i: (i, 0, 0)),
            pl.BlockSpec((1, G), lambda i: (0, 0)),
            pl.BlockSpec((G, D), lambda i: (0, 0)),
        ],
        out_shape=[
            jax.ShapeDtypeStruct((NB, B, 1), jnp.float32),
            jax.ShapeDtypeStruct((1, G), jnp.float32),
            jax.ShapeDtypeStruct((G, D), jnp.float32),
        ],
        scratch_shapes=[
            pltpu.VMEM((1, G), jnp.float32),
            pltpu.VMEM((1, G), jnp.float32),
            pltpu.VMEM((G, D), jnp.float32),
        ],
        compiler_params=pltpu.CompilerParams(
            dimension_semantics=("arbitrary",)),
    )(x, batch.reshape(NB, 1, B), wcat, bcat, w2)


def _sc_alpha(scores_pad, batch2, c):
    info = plsc.get_sparse_core_info()
    nc = info.num_cores
    mesh = plsc.VectorSubcoreMesh(core_axis_name="c", subcore_axis_name="s")

    @functools.partial(
        pl.kernel,
        mesh=mesh,
        out_type=jax.ShapeDtypeStruct((_NPAD,), jnp.float32),
        scratch_types=[
            pltpu.VMEM((_CHUNK,), jnp.float32),
            pltpu.VMEM((_CHUNK,), jnp.int32),
            pltpu.VMEM((_CHUNK,), jnp.float32),
            pltpu.VMEM((_CHUNK,), jnp.float32),
            pltpu.VMEM_SHARED((G,), jnp.float32),
            pltpu.SemaphoreType.DMA,
        ],
    )
    def k(scores_hbm, batch_hbm, c_hbm, out_hbm,
          sc_v, b_v, cg_v, a_v, c_sh, sem0):
        sid = jax.lax.axis_index("s")
        wid = sid * nc + jax.lax.axis_index("c")
        base = wid * _CHUNK

        @pl.when(sid == 0)
        def _stage():
            pltpu.sync_copy(c_hbm, c_sh)

        pltpu.sync_copy(scores_hbm.at[pl.ds(base, _CHUNK)], sc_v)
        pltpu.sync_copy(batch_hbm.at[pl.ds(base, _CHUNK)], b_v)
        plsc.subcore_barrier()

        pltpu.async_copy(c_sh.at[b_v], cg_v, sem0).wait()

        def body(j, carry):
            sl = pl.ds(j * _LANES, _LANES)
            a_v[sl] = jnp.exp(sc_v[sl] - cg_v[sl])
            return carry

        jax.lax.fori_loop(0, _CHUNK // _LANES, body, 0)
        pltpu.sync_copy(a_v, out_hbm.at[pl.ds(base, _CHUNK)])

    return k(scores_pad, batch2, c)


def kernel(x, batch, W1g, b1g, W2g, b2g, Wf, bf):
    wcat = jnp.concatenate([W1g, Wf], axis=1).astype(jnp.bfloat16)
    bcat = jnp.concatenate([b1g, bf])[None, :]
    w2 = W2g.astype(jnp.bfloat16)
    scores3, c2, emb = _run_k1(x, batch, wcat, bcat, w2)
    scores_pad = jnp.pad(scores3.reshape(N), (0, _NPAD - N))
    batch_pad = jnp.pad(batch, (0, _NPAD - N))
    alpha = _sc_alpha(scores_pad, batch_pad, c2.reshape(G))[:N]
    return emb, alpha


# speedup vs baseline: 1.1899x; 1.1899x over previous
import functools

import jax
import jax.numpy as jnp
from jax.experimental import pallas as pl
from jax.experimental.pallas import tpu as pltpu
from jax.experimental.pallas import tpu_sc as plsc

N = 50000
D = 512
H = 256
G = 128

B = 1000
NB = N // B

_LANES = 16
_KB = 13
_CHUNK = _KB * 128
_NPAD = 32 * _CHUNK


def _k1_body(x_ref, b_ref, wcat_ref, bcat_ref, w2_ref,
             scores_ref, m_ref, emb_ref,
             m_sc, s_sc, acc_sc):
    i = pl.program_id(0)

    @pl.when(i == 0)
    def _init():
        m_sc[...] = jnp.full((1, G), -jnp.inf, jnp.float32)
        s_sc[...] = jnp.zeros((1, G), jnp.float32)
        acc_sc[...] = jnp.zeros((G, D), jnp.float32)

    y = jnp.dot(x_ref[...].astype(jnp.bfloat16), wcat_ref[...],
                preferred_element_type=jnp.float32)
    y = jnp.maximum(y + bcat_ref[...], 0.0)
    g1 = y[:, :H]
    f = y[:, H:]
    score = jnp.sum(g1 * w2_ref[...], axis=1)
    scores_ref[0, 0, :] = score

    seg = b_ref[0, 0, :]
    onehot = seg[:, None] == jax.lax.broadcasted_iota(jnp.int32, (B, G), 1)
    m_old = m_sc[...][0]
    m_blk = jnp.max(jnp.where(onehot, score[:, None], -jnp.inf), axis=0)
    m_new = jnp.maximum(m_old, m_blk)
    scale = jnp.where(m_old > -jnp.inf, jnp.exp(m_old - m_new), 0.0)
    e = jnp.where(onehot, jnp.exp(score[:, None] - m_new[None, :]), 0.0)
    s_sc[...] = (s_sc[...][0] * scale + jnp.sum(e, axis=0))[None, :]
    acc_sc[...] = acc_sc[...] * scale[:, None] + jax.lax.dot_general(
        e.astype(jnp.bfloat16), f.astype(jnp.bfloat16),
        (((0,), (0,)), ((), ())), preferred_element_type=jnp.float32)
    m_sc[...] = m_new[None, :]

    @pl.when(i == NB - 1)
    def _fin():
        sden = s_sc[...]
        m_ref[...] = m_sc[...] + jnp.log(sden)
        scol = sden[0][:, None]
        emb_ref[...] = jnp.where(scol > 0.0, acc_sc[...] / scol, 0.0)


def _run_k1(x, batch, wcat, bcat, w2):
    return pl.pallas_call(
        _k1_body,
        grid=(NB,),
        in_specs=[
            pl.BlockSpec((B, D), lambda i: (i, 0)),
            pl.BlockSpec((1, 1, B), lambda i: (i, 0, 0)),
            pl.BlockSpec((D, H + D), lambda i: (0, 0)),
            pl.BlockSpec((1, H + D), lambda i: (0, 0)),
            pl.BlockSpec((1, H), lambda i: (0, 0)),
        ],
        out_specs=[
            pl.BlockSpec((1, 1, B), lambda i: (i, 0, 0)),
            pl.BlockSpec((1, G), lambda i: (0, 0)),
            pl.BlockSpec((G, D), lambda i: (0, 0)),
        ],
        out_shape=[
            jax.ShapeDtypeStruct((NB, 1, B), jnp.float32),
            jax.ShapeDtypeStruct((1, G), jnp.float32),
            jax.ShapeDtypeStruct((G, D), jnp.float32),
        ],
        scratch_shapes=[
            pltpu.VMEM((1, G), jnp.float32),
            pltpu.VMEM((1, G), jnp.float32),
            pltpu.VMEM((G, D), jnp.float32),
        ],
        compiler_params=pltpu.CompilerParams(
            dimension_semantics=("arbitrary",)),
    )(x, batch.reshape(NB, 1, B), wcat, bcat, w2)


def _sc_alpha(scores_pad, batch2, c):
    info = plsc.get_sparse_core_info()
    nc = info.num_cores
    mesh = plsc.VectorSubcoreMesh(core_axis_name="c", subcore_axis_name="s")

    @functools.partial(
        pl.kernel,
        mesh=mesh,
        out_type=jax.ShapeDtypeStruct((_NPAD,), jnp.float32),
        scratch_types=[
            pltpu.VMEM((_CHUNK,), jnp.float32),
            pltpu.VMEM((_CHUNK,), jnp.int32),
            pltpu.VMEM((_CHUNK,), jnp.float32),
            pltpu.VMEM((_CHUNK,), jnp.float32),
            pltpu.VMEM_SHARED((G,), jnp.float32),
            pltpu.SemaphoreType.DMA,
        ],
    )
    def k(scores_hbm, batch_hbm, c_hbm, out_hbm,
          sc_v, b_v, cg_v, a_v, c_sh, sem0):
        sid = jax.lax.axis_index("s")
        wid = sid * nc + jax.lax.axis_index("c")
        base = wid * _CHUNK

        @pl.when(sid == 0)
        def _stage():
            pltpu.sync_copy(c_hbm, c_sh)

        pltpu.sync_copy(scores_hbm.at[pl.ds(base, _CHUNK)], sc_v)
        pltpu.sync_copy(batch_hbm.at[pl.ds(base, _CHUNK)], b_v)
        plsc.subcore_barrier()

        pltpu.async_copy(c_sh.at[b_v], cg_v, sem0).wait()

        def body(j, carry):
            sl = pl.ds(j * _LANES, _LANES)
            a_v[sl] = jnp.exp(sc_v[sl] - cg_v[sl])
            return carry

        jax.lax.fori_loop(0, _CHUNK // _LANES, body, 0)
        pltpu.sync_copy(a_v, out_hbm.at[pl.ds(base, _CHUNK)])

    return k(scores_pad, batch2, c)


def kernel(x, batch, W1g, b1g, W2g, b2g, Wf, bf):
    wcat = jnp.concatenate([W1g, Wf], axis=1).astype(jnp.bfloat16)
    bcat = jnp.concatenate([b1g, bf])[None, :]
    w2 = W2g.reshape(1, H)
    scores3, c2, emb = _run_k1(x, batch, wcat, bcat, w2)
    scores_pad = jnp.pad(scores3.reshape(N), (0, _NPAD - N))
    batch_pad = jnp.pad(batch, (0, _NPAD - N))
    alpha = _sc_alpha(scores_pad, batch_pad, c2.reshape(G))[:N]
    return emb, alpha


# speedup vs baseline: 1.1926x; 1.0023x over previous
import functools

import jax
import jax.numpy as jnp
from jax.experimental import pallas as pl
from jax.experimental.pallas import tpu as pltpu
from jax.experimental.pallas import tpu_sc as plsc

N = 50000
D = 512
H = 256
G = 128

B = 2000
NB = N // B

_LANES = 16
_KB = 13
_CHUNK = _KB * 128
_NPAD = 32 * _CHUNK


def _k1_body(x_ref, b_ref, wcat_ref, bcat_ref, w2_ref,
             scores_ref, m_ref, emb_ref,
             m_sc, s_sc, acc_sc):
    i = pl.program_id(0)

    @pl.when(i == 0)
    def _init():
        m_sc[...] = jnp.full((1, G), -jnp.inf, jnp.float32)
        s_sc[...] = jnp.zeros((1, G), jnp.float32)
        acc_sc[...] = jnp.zeros((G, D), jnp.float32)

    y = jnp.dot(x_ref[...].astype(jnp.bfloat16), wcat_ref[...],
                preferred_element_type=jnp.float32)
    y = jnp.maximum(y + bcat_ref[...], 0.0)
    g1 = y[:, :H]
    f = y[:, H:]
    score = jnp.sum(g1 * w2_ref[...], axis=1)
    scores_ref[0, 0, :] = score

    seg = b_ref[0, 0, :]
    onehot = seg[:, None] == jax.lax.broadcasted_iota(jnp.int32, (B, G), 1)
    m_old = m_sc[...][0]
    m_blk = jnp.max(jnp.where(onehot, score[:, None], -jnp.inf), axis=0)
    m_new = jnp.maximum(m_old, m_blk)
    scale = jnp.where(m_old > -jnp.inf, jnp.exp(m_old - m_new), 0.0)
    e = jnp.where(onehot, jnp.exp(score[:, None] - m_new[None, :]), 0.0)
    s_sc[...] = (s_sc[...][0] * scale + jnp.sum(e, axis=0))[None, :]
    acc_sc[...] = acc_sc[...] * scale[:, None] + jax.lax.dot_general(
        e.astype(jnp.bfloat16), f.astype(jnp.bfloat16),
        (((0,), (0,)), ((), ())), preferred_element_type=jnp.float32)
    m_sc[...] = m_new[None, :]

    @pl.when(i == NB - 1)
    def _fin():
        sden = s_sc[...]
        m_ref[...] = m_sc[...] + jnp.log(sden)
        scol = sden[0][:, None]
        emb_ref[...] = jnp.where(scol > 0.0, acc_sc[...] / scol, 0.0)


def _run_k1(x, batch, wcat, bcat, w2):
    return pl.pallas_call(
        _k1_body,
        grid=(NB,),
        in_specs=[
            pl.BlockSpec((B, D), lambda i: (i, 0)),
            pl.BlockSpec((1, 1, B), lambda i: (i, 0, 0)),
            pl.BlockSpec((D, H + D), lambda i: (0, 0)),
            pl.BlockSpec((1, H + D), lambda i: (0, 0)),
            pl.BlockSpec((1, H), lambda i: (0, 0)),
        ],
        out_specs=[
            pl.BlockSpec((1, 1, B), lambda i: (i, 0, 0)),
            pl.BlockSpec((1, G), lambda i: (0, 0)),
            pl.BlockSpec((G, D), lambda i: (0, 0)),
        ],
        out_shape=[
            jax.ShapeDtypeStruct((NB, 1, B), jnp.float32),
            jax.ShapeDtypeStruct((1, G), jnp.float32),
            jax.ShapeDtypeStruct((G, D), jnp.float32),
        ],
        scratch_shapes=[
            pltpu.VMEM((1, G), jnp.float32),
            pltpu.VMEM((1, G), jnp.float32),
            pltpu.VMEM((G, D), jnp.float32),
        ],
        compiler_params=pltpu.CompilerParams(
            dimension_semantics=("arbitrary",)),
    )(x, batch.reshape(NB, 1, B), wcat, bcat, w2)


def _sc_alpha(scores_pad, batch2, c):
    info = plsc.get_sparse_core_info()
    nc = info.num_cores
    mesh = plsc.VectorSubcoreMesh(core_axis_name="c", subcore_axis_name="s")

    @functools.partial(
        pl.kernel,
        mesh=mesh,
        out_type=jax.ShapeDtypeStruct((_NPAD,), jnp.float32),
        scratch_types=[
            pltpu.VMEM((_CHUNK,), jnp.float32),
            pltpu.VMEM((_CHUNK,), jnp.int32),
            pltpu.VMEM((_CHUNK,), jnp.float32),
            pltpu.VMEM((_CHUNK,), jnp.float32),
            pltpu.VMEM_SHARED((G,), jnp.float32),
            pltpu.SemaphoreType.DMA,
        ],
    )
    def k(scores_hbm, batch_hbm, c_hbm, out_hbm,
          sc_v, b_v, cg_v, a_v, c_sh, sem0):
        sid = jax.lax.axis_index("s")
        wid = sid * nc + jax.lax.axis_index("c")
        base = wid * _CHUNK

        @pl.when(sid == 0)
        def _stage():
            pltpu.sync_copy(c_hbm, c_sh)

        pltpu.sync_copy(scores_hbm.at[pl.ds(base, _CHUNK)], sc_v)
        pltpu.sync_copy(batch_hbm.at[pl.ds(base, _CHUNK)], b_v)
        plsc.subcore_barrier()

        pltpu.async_copy(c_sh.at[b_v], cg_v, sem0).wait()

        def body(j, carry):
            sl = pl.ds(j * _LANES, _LANES)
            a_v[sl] = jnp.exp(sc_v[sl] - cg_v[sl])
            return carry

        jax.lax.fori_loop(0, _CHUNK // _LANES, body, 0)
        pltpu.sync_copy(a_v, out_hbm.at[pl.ds(base, _CHUNK)])

    return k(scores_pad, batch2, c)


def kernel(x, batch, W1g, b1g, W2g, b2g, Wf, bf):
    wcat = jnp.concatenate([W1g, Wf], axis=1).astype(jnp.bfloat16)
    bcat = jnp.concatenate([b1g, bf])[None, :]
    w2 = W2g.reshape(1, H)
    scores3, c2, emb = _run_k1(x, batch, wcat, bcat, w2)
    scores_pad = jnp.pad(scores3.reshape(N), (0, _NPAD - N))
    batch_pad = jnp.pad(batch, (0, _NPAD - N))
    alpha = _sc_alpha(scores_pad, batch_pad, c2.reshape(G))[:N]
    return emb, alpha


# speedup vs baseline: 1.2220x; 1.0247x over previous
import functools

import jax
import jax.numpy as jnp
from jax.experimental import pallas as pl
from jax.experimental.pallas import tpu as pltpu
from jax.experimental.pallas import tpu_sc as plsc

N = 50000
D = 512
H = 256
G = 128

B = 2000
NB = N // B

_LANES = 16
_CHUNK = 1568
_LAST = N - _CHUNK


def _k1_body(x_ref, b_ref, wcat_ref, bcat_ref, w2_ref,
             scores_ref, m_ref, emb_ref,
             m_sc, s_sc, acc_sc):
    i = pl.program_id(0)

    @pl.when(i == 0)
    def _init():
        m_sc[...] = jnp.full((1, G), -jnp.inf, jnp.float32)
        s_sc[...] = jnp.zeros((1, G), jnp.float32)
        acc_sc[...] = jnp.zeros((G, D), jnp.float32)

    y = jnp.dot(x_ref[...].astype(jnp.bfloat16), wcat_ref[...],
                preferred_element_type=jnp.float32)
    y = jnp.maximum(y + bcat_ref[...], 0.0)
    g1 = y[:, :H]
    f = y[:, H:]
    score = jnp.sum(g1 * w2_ref[...], axis=1)
    scores_ref[0, 0, :] = score

    seg = b_ref[0, 0, :]
    onehot = seg[:, None] == jax.lax.broadcasted_iota(jnp.int32, (B, G), 1)
    m_old = m_sc[...][0]
    m_blk = jnp.max(jnp.where(onehot, score[:, None], -jnp.inf), axis=0)
    m_new = jnp.maximum(m_old, m_blk)
    scale = jnp.where(m_old > -jnp.inf, jnp.exp(m_old - m_new), 0.0)
    e = jnp.where(onehot, jnp.exp(score[:, None] - m_new[None, :]), 0.0)
    s_sc[...] = (s_sc[...][0] * scale + jnp.sum(e, axis=0))[None, :]
    acc_sc[...] = acc_sc[...] * scale[:, None] + jax.lax.dot_general(
        e.astype(jnp.bfloat16), f.astype(jnp.bfloat16),
        (((0,), (0,)), ((), ())), preferred_element_type=jnp.float32)
    m_sc[...] = m_new[None, :]

    @pl.when(i == NB - 1)
    def _fin():
        sden = s_sc[...]
        m_ref[...] = m_sc[...] + jnp.log(sden)
        scol = sden[0][:, None]
        emb_ref[...] = jnp.where(scol > 0.0, acc_sc[...] / scol, 0.0)


def _run_k1(x, batch, wcat, bcat, w2):
    return pl.pallas_call(
        _k1_body,
        grid=(NB,),
        in_specs=[
            pl.BlockSpec((B, D), lambda i: (i, 0)),
            pl.BlockSpec((1, 1, B), lambda i: (i, 0, 0)),
            pl.BlockSpec((D, H + D), lambda i: (0, 0)),
            pl.BlockSpec((1, H + D), lambda i: (0, 0)),
            pl.BlockSpec((1, H), lambda i: (0, 0)),
        ],
        out_specs=[
            pl.BlockSpec((1, 1, B), lambda i: (i, 0, 0)),
            pl.BlockSpec((1, G), lambda i: (0, 0)),
            pl.BlockSpec((G, D), lambda i: (0, 0)),
        ],
        out_shape=[
            jax.ShapeDtypeStruct((NB, 1, B), jnp.float32),
            jax.ShapeDtypeStruct((1, G), jnp.float32),
            jax.ShapeDtypeStruct((G, D), jnp.float32),
        ],
        scratch_shapes=[
            pltpu.VMEM((1, G), jnp.float32),
            pltpu.VMEM((1, G), jnp.float32),
            pltpu.VMEM((G, D), jnp.float32),
        ],
        compiler_params=pltpu.CompilerParams(
            dimension_semantics=("arbitrary",)),
    )(x, batch.reshape(NB, 1, B), wcat, bcat, w2)


def _sc_alpha(scores, batch, c):
    info = plsc.get_sparse_core_info()
    nc = info.num_cores
    mesh = plsc.VectorSubcoreMesh(core_axis_name="c", subcore_axis_name="s")

    @functools.partial(
        pl.kernel,
        mesh=mesh,
        out_type=jax.ShapeDtypeStruct((N,), jnp.float32),
        scratch_types=[
            pltpu.VMEM((_CHUNK,), jnp.float32),
            pltpu.VMEM((_CHUNK,), jnp.int32),
            pltpu.VMEM((_CHUNK,), jnp.float32),
            pltpu.VMEM((_CHUNK,), jnp.float32),
            pltpu.VMEM_SHARED((G,), jnp.float32),
            pltpu.SemaphoreType.DMA,
        ],
    )
    def k(scores_hbm, batch_hbm, c_hbm, out_hbm,
          sc_v, b_v, cg_v, a_v, c_sh, sem0):
        sid = jax.lax.axis_index("s")
        wid = sid * nc + jax.lax.axis_index("c")
        base = jnp.minimum(wid * _CHUNK, _LAST)

        @pl.when(sid == 0)
        def _stage():
            pltpu.sync_copy(c_hbm, c_sh)

        pltpu.sync_copy(scores_hbm.at[pl.ds(base, _CHUNK)], sc_v)
        pltpu.sync_copy(batch_hbm.at[pl.ds(base, _CHUNK)], b_v)
        plsc.subcore_barrier()

        pltpu.async_copy(c_sh.at[b_v], cg_v, sem0).wait()

        def body(j, carry):
            sl = pl.ds(j * _LANES, _LANES)
            a_v[sl] = jnp.exp(sc_v[sl] - cg_v[sl])
            return carry

        jax.lax.fori_loop(0, _CHUNK // _LANES, body, 0)
        pltpu.sync_copy(a_v, out_hbm.at[pl.ds(base, _CHUNK)])

    return k(scores, batch, c)


def kernel(x, batch, W1g, b1g, W2g, b2g, Wf, bf):
    wcat = jnp.concatenate([W1g, Wf], axis=1).astype(jnp.bfloat16)
    bcat = jnp.concatenate([b1g, bf])[None, :]
    w2 = W2g.reshape(1, H)
    scores3, c2, emb = _run_k1(x, batch, wcat, bcat, w2)
    alpha = _sc_alpha(scores3.reshape(N), batch, c2.reshape(G))
    return emb, alpha
